# Initial kernel scaffold; baseline (speedup 1.0000x reference)
import jax
import jax.numpy as jnp
from jax.experimental import pallas as pl


def kernel(x, edge_index, params):
    raise NotImplementedError("write your pallas kernel here")


# speedup vs baseline: 3.7327x; 3.7327x over previous
import functools
import math

import jax
import jax.numpy as jnp
import numpy as np
from jax import lax
from jax.experimental import pallas as pl
from jax.experimental.pallas import tpu as pltpu
from jax.experimental.pallas import tpu_sc as plsc

_H = 4
_IN_CH = 128
_HIDDEN = [32, 64, 128, 256]
_RATIOS = [0.8, 0.6, 0.4]
_NWORK = 32
_CHUNK = 128
_BN = 256


def _rup(v, m):
    return (v + m - 1) // m * m


def _prep_body(F, x_ref, wt_ref, ab_ref, hs_ref, hd_ref):
    h = jnp.dot(x_ref[...], wt_ref[...], preferred_element_type=jnp.float32)
    asad = jnp.dot(h, ab_ref[...], preferred_element_type=jnp.float32)
    hs_ref[:, :F] = h
    hs_ref[:, F:] = asad[:, :16]
    hd_ref[...] = asad[:, 16:]


def _prep(xp, W, asrc, adst):
    Npad, fin = xp.shape
    F = W.shape[0]
    C = F // _H
    head = np.arange(F) // C
    sel = (head[:, None] == np.arange(16)[None, :]).astype(np.float32)
    ab = jnp.concatenate([asrc.reshape(F)[:, None] * sel,
                          adst.reshape(F)[:, None] * sel], axis=1)
    grid = (Npad // _BN,)
    hs, hd = pl.pallas_call(
        functools.partial(_prep_body, F),
        grid=grid,
        in_specs=[pl.BlockSpec((_BN, fin), lambda i: (i, 0)),
                  pl.BlockSpec((fin, F), lambda i: (0, 0)),
                  pl.BlockSpec((F, 32), lambda i: (0, 0))],
        out_specs=[pl.BlockSpec((_BN, F + 16), lambda i: (i, 0)),
                   pl.BlockSpec((_BN, 16), lambda i: (i, 0))],
        out_shape=[jax.ShapeDtypeStruct((Npad, F + 16), jnp.float32),
                   jax.ShapeDtypeStruct((Npad, 16), jnp.float32)],
    )(xp, W.T, ab)
    return hs, hd


def _sc_edge(F, Npad, hs, hd, srcl, dstl, asmax, nsplit=0):
    FP = F + 16
    NB = F // 16
    C = F // _H
    Epad = srcl.shape[0]
    nwork = 16 if nsplit else _NWORK
    EW = Epad // nwork
    nch = EW // _CHUNK
    accrows = _rup(nsplit + 16, 256) if nsplit else Npad
    RPS = accrows // 16
    ZB = 16
    mesh = plsc.VectorSubcoreMesh(core_axis_name="c", subcore_axis_name="s",
                                  num_cores=2, num_subcores=16)

    @functools.partial(
        pl.kernel,
        out_type=jax.ShapeDtypeStruct((2, accrows, FP), jnp.float32),
        mesh=mesh,
        compiler_params=pltpu.CompilerParams(use_tc_tiling_on_sc=False,
                                             needs_layout_passes=False),
        scratch_types=[
            pltpu.VMEM((_CHUNK,), jnp.int32),
            pltpu.VMEM((_CHUNK,), jnp.int32),
            pltpu.VMEM((_CHUNK, FP), jnp.float32),
            pltpu.VMEM((_CHUNK, 16), jnp.float32),
            pltpu.VMEM((_CHUNK, FP), jnp.float32),
            pltpu.VMEM((16,), jnp.float32),
            pltpu.VMEM((ZB, FP), jnp.float32),
            pltpu.VMEM_SHARED((accrows, FP), jnp.float32),
            pltpu.SemaphoreType.DMA,
            pltpu.SemaphoreType.DMA,
        ],
    )
    def k(hs_hbm, hd_hbm, src_hbm, dst_hbm, mx_hbm, out_hbm,
          isrc, idst, rows, hdr, obuf, mxv, zv, accsh, sem1, sem2):
        cid = lax.axis_index("c")
        sid = lax.axis_index("s")
        zero16 = jnp.zeros((16,), jnp.float32)

        @pl.loop(0, ZB)
        def _(r):
            @pl.loop(0, FP // 16)
            def _(cb):
                zv[r, pl.ds(cb * 16, 16)] = zero16

        @pl.loop(0, RPS // ZB)
        def _(z):
            pltpu.sync_copy(zv, accsh.at[pl.ds(sid * RPS + z * ZB, ZB)])

        pltpu.sync_copy(mx_hbm, mxv)
        plsc.subcore_barrier()

        wid = sid if nsplit else sid * 2 + cid
        iota = lax.iota(jnp.int32, 16)

        @pl.loop(0, nch)
        def _(t):
            base = wid * EW + t * _CHUNK
            pltpu.sync_copy(src_hbm.at[pl.ds(base, _CHUNK)], isrc)
            pltpu.sync_copy(dst_hbm.at[pl.ds(base, _CHUNK)], idst)
            g1 = pltpu.async_copy(hs_hbm.at[isrc], rows, sem1)
            g2 = pltpu.async_copy(hd_hbm.at[idst], hdr, sem2)
            g1.wait()
            g2.wait()
            if nsplit:
                off = cid * nsplit

                @pl.loop(0, _CHUNK // 16)
                def _(q):
                    v = idst[pl.ds(q * 16, 16)] - off
                    inb = (v >= 0) & (v < nsplit)
                    idst[pl.ds(q * 16, 16)] = jnp.where(inb, v, nsplit)
            mx = mxv[...]

            @pl.loop(0, _CHUNK)
            def _(ed):
                s16 = rows[ed, pl.ds(F, 16)]
                t16 = hdr[ed, pl.ds(0, 16)]
                e = s16 + t16
                e = jnp.where(e > 0, e, e * 0.2)
                cm = mx + t16
                cm = jnp.where(cm > 0, cm, cm * 0.2)
                ee = jnp.exp(e - cm)
                obuf[ed, pl.ds(F, 16)] = ee
                svals = [jnp.max(jnp.where(iota == hd, ee, -1e30))
                         for hd in range(_H)]
                zv16 = ee * 0.0
                bvecs = [sv + zv16 for sv in svals]
                for fb in range(NB):
                    lo = fb * 16
                    h0 = lo // C
                    h1 = (lo + 15) // C
                    if h0 == h1:
                        bc = bvecs[h0]
                    else:
                        bc = jnp.where(iota < (h1 * C - lo), bvecs[h0], bvecs[h1])
                    obuf[ed, pl.ds(fb * 16, 16)] = rows[ed, pl.ds(fb * 16, 16)] * bc

            pltpu.sync_copy(obuf, accsh.at[idst], add=True)

        plsc.subcore_barrier()
        pltpu.sync_copy(accsh.at[pl.ds(sid * RPS, RPS)],
                        out_hbm.at[cid, pl.ds(sid * RPS, RPS)])

    return k(hs, hd, srcl, dstl, asmax)


def _epi_body(F, has_ln, p_ref, q_ref, qmat_ref, b_ref, lnw_ref, lnb_ref,
              wsc_ref, o_ref, sc_ref):
    acc = p_ref[...] + q_ref[...]
    denF = jnp.dot(acc[:, F:], qmat_ref[...], preferred_element_type=jnp.float32)
    out = acc[:, :F] / (denF + 1e-16) + b_ref[...]
    if has_ln:
        mu = jnp.mean(out, axis=1, keepdims=True)
        var = jnp.mean((out - mu) ** 2, axis=1, keepdims=True)
        out = (out - mu) * jax.lax.rsqrt(var + 1e-5) * lnw_ref[...] + lnb_ref[...]
        out = jnp.where(out > 0, out, out * 0.2)
    o_ref[...] = out
    if sc_ref is not None:
        sc_ref[...] = jnp.tanh(jnp.dot(out, wsc_ref[...],
                                       preferred_element_type=jnp.float32))


def _epi(F, partials, b, lnw, lnb, pool_w):
    Npad = partials.shape[1]
    FP = F + 16
    has_ln = lnw is not None
    C = F // _H
    head = np.arange(F) // C
    qmat = jnp.asarray((np.arange(16)[:, None] == head[None, :]).astype(np.float32))
    if not has_ln:
        lnw = jnp.zeros((1, F), jnp.float32)
        lnb = jnp.zeros((1, F), jnp.float32)
    has_sc = pool_w is not None
    if has_sc:
        wn = pool_w / (jnp.linalg.norm(pool_w) + 1e-16)
        wsc = jnp.zeros((F, 128), jnp.float32).at[:, 0].set(wn)
    else:
        wsc = jnp.zeros((8, 128), jnp.float32)
    grid = (Npad // _BN,)
    out_specs = [pl.BlockSpec((_BN, F), lambda i: (i, 0))]
    out_shape = [jax.ShapeDtypeStruct((Npad, F), jnp.float32)]
    if has_sc:
        out_specs.append(pl.BlockSpec((_BN, 128), lambda i: (i, 0)))
        out_shape.append(jax.ShapeDtypeStruct((Npad, 128), jnp.float32))

    def body(p_ref, q_ref, qmat_ref, b_ref, lnw_ref, lnb_ref, wsc_ref,
             o_ref, sc_ref=None):
        _epi_body(F, has_ln, p_ref, q_ref, qmat_ref, b_ref, lnw_ref, lnb_ref,
                  wsc_ref, o_ref, sc_ref)

    res = pl.pallas_call(
        body,
        grid=grid,
        in_specs=[pl.BlockSpec((_BN, FP), lambda i: (i, 0)),
                  pl.BlockSpec((_BN, FP), lambda i: (i, 0)),
                  pl.BlockSpec((16, F), lambda i: (0, 0)),
                  pl.BlockSpec((1, F), lambda i: (0, 0)),
                  pl.BlockSpec((1, F), lambda i: (0, 0)),
                  pl.BlockSpec((1, F), lambda i: (0, 0)),
                  pl.BlockSpec(wsc.shape, lambda i: (0, 0))],
        out_specs=out_specs,
        out_shape=out_shape,
    )(partials[0], partials[1], qmat, b.reshape(1, F), lnw.reshape(1, F),
      lnb.reshape(1, F), wsc)
    return (res[0], res[1][:, 0]) if has_sc else (res[0], None)


def _gat_block(xp, N, srcl, dstl, params, pfx, final=False, pool_w=None,
               nsplit=0):
    F = params[pfx + "_W"].shape[0]
    Npad = xp.shape[0]
    hs, hd = _prep(xp, params[pfx + "_W"], params[pfx + "_asrc"],
                   params[pfx + "_adst"])
    asmax = jnp.max(hs[:N, F:], axis=0)
    asmax = asmax + jnp.where(jnp.arange(16) < _H, 0.0, 1e9).astype(jnp.float32)
    partials = _sc_edge(F, Npad, hs, hd, srcl, dstl, asmax, nsplit=nsplit)
    if nsplit:
        acc = partials[:, :nsplit, :].reshape(2 * nsplit, F + 16)
        acc = jnp.pad(acc, ((0, Npad - 2 * nsplit), (0, 0)))
        partials = jnp.stack([acc, jnp.zeros_like(acc)])
    lnw = None if final else params[pfx + "_ln_w"]
    lnb = None if final else params[pfx + "_ln_b"]
    return _epi(F, partials, params[pfx + "_b"], lnw, lnb, pool_w)


def _pad_edges(src, dst, N):
    E = src.shape[0]
    Epad = _rup(E, _NWORK * _CHUNK)
    sent = jnp.int32(N)
    pad = jnp.full((Epad - E,), sent, jnp.int32)
    return jnp.concatenate([src.astype(jnp.int32), pad]), \
        jnp.concatenate([dst.astype(jnp.int32), pad])


def kernel(x, edge_index, params):
    N0 = x.shape[0]
    E = edge_index.shape[1]
    dt = edge_index.dtype

    Ns = [N0]
    for r in _RATIOS:
        Ns.append(int(math.ceil(r * Ns[-1])))

    loop0 = jnp.arange(N0, dtype=dt)
    src0 = jnp.concatenate([edge_index[0], loop0])
    dst0 = jnp.concatenate([edge_index[1], loop0])
    srcl0, dstl0 = _pad_edges(src0, dst0, N0)

    cur_ei = edge_index
    cur_x = x
    skips = []
    for i in range(3):
        N = Ns[i]
        Npad = _rup(N + 1, _BN)
        xp = jnp.zeros((Npad, cur_x.shape[1]), jnp.float32).at[:N].set(cur_x)
        if i == 0:
            srcl, dstl = srcl0, dstl0
        else:
            loop = jnp.arange(N, dtype=dt)
            srcl, dstl = _pad_edges(jnp.concatenate([cur_ei[0], loop]),
                                    jnp.concatenate([cur_ei[1], loop]), N)
        h, score = _gat_block(xp, N, srcl, dstl, params, "enc%d" % i,
                              pool_w=params["enc%d_pool_w" % i])
        h = h[:N]
        score = score[:N]
        k = Ns[i + 1]
        vals, perm = lax.top_k(score, k)
        perm = perm.astype(dt)
        kept = jnp.zeros((N,), bool).at[perm].set(True)
        ei0 = jnp.clip(cur_ei[0], 0, N - 1)
        ei1 = jnp.clip(cur_ei[1], 0, N - 1)
        mask = (cur_ei[0] < N) & (cur_ei[1] < N) & kept[ei0] & kept[ei1]
        remap = jnp.full((N,), k, dtype=dt).at[perm].set(jnp.arange(k, dtype=dt))
        sent = jnp.asarray(k, dtype=dt)
        new_ei = jnp.where(mask[None, :], remap[jnp.stack([ei0, ei1])], sent)
        x_new = h[perm] * vals[:, None]
        skips.append((h, cur_ei, perm, remap))
        cur_x, cur_ei = x_new, new_ei

    N = Ns[3]
    Npad = _rup(N + 1, _BN)
    xp = jnp.zeros((Npad, cur_x.shape[1]), jnp.float32).at[:N].set(cur_x)
    loop = jnp.arange(N, dtype=dt)
    srcl, dstl = _pad_edges(jnp.concatenate([cur_ei[0], loop]),
                            jnp.concatenate([cur_ei[1], loop]), N)
    cur_x, _ = _gat_block(xp, N, srcl, dstl, params, "bot")
    cur_x = cur_x[:N]

    for j, (h_skip, e_skip, perm, remap) in enumerate(reversed(skips)):
        N = h_skip.shape[0]
        k = cur_x.shape[0]
        up = jnp.zeros((N, cur_x.shape[1]), jnp.float32).at[perm].set(cur_x)
        cat = jnp.concatenate([up, h_skip], axis=-1)
        Npad = _rup(N + 1, _BN)
        xp = jnp.zeros((Npad, cat.shape[1]), jnp.float32).at[:N].set(cat)
        loop = jnp.arange(N, dtype=dt)
        srcl, dstl = _pad_edges(jnp.concatenate([e_skip[0], loop]),
                                jnp.concatenate([e_skip[1], loop]), N)
        cur_x, _ = _gat_block(xp, N, srcl, dstl, params, "dec%d" % j)
        cur_x = cur_x[:N]

    Npad = _rup(N0 + 1, _BN)
    xp = jnp.zeros((Npad, cur_x.shape[1]), jnp.float32).at[:N0].set(cur_x)
    nsplit = _rup((N0 + 2) // 2, 16)
    out, _ = _gat_block(xp, N0, srcl0, dstl0, params, "out", final=True,
                        nsplit=nsplit)
    return out[:N0, :_IN_CH]
